# baseline (speedup 1.0000x reference)
import functools

import jax
import jax.numpy as jnp
from jax.experimental import pallas as pl
from jax.experimental.pallas import tpu as pltpu

Q = 1024
D = 128
K = 100000
BK = 2000
KBLOCKS = K // BK


def _produce(lat_ref, c_ref, dst):
    lat = lat_ref[...]
    blk = c_ref[...]
    abm2 = jax.lax.dot_general(
        lat * -2.0, blk, (((1,), (1,)), ((), ())),
        preferred_element_type=jnp.float32,
    )
    a2 = jnp.sum(lat * lat, axis=1, keepdims=True)
    b2 = jnp.sum(blk * blk, axis=1)[None, :]
    dst[...] = (a2 + b2) + abm2


def _consume(j, src, best_val, best_idx):
    d2 = src[...]
    bmin = jnp.min(d2, axis=1, keepdims=True)
    iota = jax.lax.broadcasted_iota(jnp.int32, (1, BK), 1).astype(jnp.float32)
    barg = (
        jnp.min(
            jnp.where(d2 == bmin, iota, jnp.float32(BK)), axis=1, keepdims=True
        ).astype(jnp.int32)
        + j * BK
    )

    @pl.when(j == 0)
    def _init():
        best_val[...] = bmin
        best_idx[...] = barg

    @pl.when(j > 0)
    def _update():
        improve = bmin < best_val[...]
        best_val[...] = jnp.where(improve, bmin, best_val[...])
        best_idx[...] = jnp.where(improve, barg, best_idx[...])


def _nearest_kernel(lat_ref, c_ref, out_ref, d2a, d2b, best_val, best_idx):
    k = pl.program_id(0)

    @pl.when(k % 2 == 0)
    def _even():
        @pl.when(k < KBLOCKS)
        def _p():
            _produce(lat_ref, c_ref, d2a)

        @pl.when(k > 0)
        def _c():
            _consume(k - 1, d2b, best_val, best_idx)

    @pl.when(k % 2 == 1)
    def _odd():
        @pl.when(k < KBLOCKS)
        def _p():
            _produce(lat_ref, c_ref, d2b)

        @pl.when(k > 0)
        def _c():
            _consume(k - 1, d2a, best_val, best_idx)

    @pl.when(k == KBLOCKS)
    def _emit():
        out_ref[...] = best_idx[...]


@functools.partial(jax.jit, static_argnames=())
def kernel(latent, coords1, coords2):
    del coords2
    out = pl.pallas_call(
        _nearest_kernel,
        grid=(KBLOCKS + 1,),
        in_specs=[
            pl.BlockSpec((Q, D), lambda k: (0, 0)),
            pl.BlockSpec((BK, D), lambda k: (jnp.minimum(k, KBLOCKS - 1), 0)),
        ],
        out_specs=pl.BlockSpec((Q, 1), lambda k: (0, 0)),
        out_shape=jax.ShapeDtypeStruct((Q, 1), jnp.int32),
        scratch_shapes=[
            pltpu.VMEM((Q, BK), jnp.float32),
            pltpu.VMEM((Q, BK), jnp.float32),
            pltpu.VMEM((Q, 1), jnp.float32),
            pltpu.VMEM((Q, 1), jnp.int32),
        ],
    )(latent, coords1)
    return out.reshape(Q)


# speedup vs baseline: 1.0923x; 1.0923x over previous
import functools

import jax
import jax.numpy as jnp
from jax.experimental import pallas as pl
from jax.experimental.pallas import tpu as pltpu

Q = 1024
D = 128
K = 100000
BK = 4000
KBLOCKS = K // BK


def _nearest_kernel(lat_ref, c_ref, out_ref, best_val, best_idx):
    k = pl.program_id(0)
    lat = lat_ref[...]
    blk = c_ref[...]
    abm2 = jax.lax.dot_general(
        lat * -2.0, blk, (((1,), (1,)), ((), ())),
        preferred_element_type=jnp.float32,
    )
    a2 = jnp.sum(lat * lat, axis=1, keepdims=True)
    b2 = jnp.sum(blk * blk, axis=1)[None, :]
    d2 = (a2 + b2) + abm2
    bmin = jnp.min(d2, axis=1, keepdims=True)
    iota = jax.lax.broadcasted_iota(jnp.int32, (1, BK), 1).astype(jnp.float32)
    barg = (
        jnp.min(
            jnp.where(d2 == bmin, iota, jnp.float32(BK)), axis=1, keepdims=True
        ).astype(jnp.int32)
        + k * BK
    )

    @pl.when(k == 0)
    def _init():
        best_val[...] = bmin
        best_idx[...] = barg

    @pl.when(k > 0)
    def _update():
        improve = bmin < best_val[...]
        best_val[...] = jnp.where(improve, bmin, best_val[...])
        best_idx[...] = jnp.where(improve, barg, best_idx[...])

    @pl.when(k == KBLOCKS - 1)
    def _emit():
        out_ref[...] = best_idx[...]


@functools.partial(jax.jit, static_argnames=())
def kernel(latent, coords1, coords2):
    del coords2
    out = pl.pallas_call(
        _nearest_kernel,
        grid=(KBLOCKS,),
        in_specs=[
            pl.BlockSpec((Q, D), lambda k: (0, 0)),
            pl.BlockSpec((BK, D), lambda k: (k, 0)),
        ],
        out_specs=pl.BlockSpec((Q, 1), lambda k: (0, 0)),
        out_shape=jax.ShapeDtypeStruct((Q, 1), jnp.int32),
        scratch_shapes=[
            pltpu.VMEM((Q, 1), jnp.float32),
            pltpu.VMEM((Q, 1), jnp.int32),
        ],
    )(latent, coords1)
    return out.reshape(Q)
